# baseline (speedup 1.0000x reference)
import functools

import jax
import jax.numpy as jnp
from jax import lax
from jax.experimental import pallas as pl
from jax.experimental.pallas import tpu as pltpu
from jax.experimental.pallas import tpu_sc as plsc

T = 2048
D = 1024
E = 8
F = 2048
C = (2 * T) // E

_mesh = plsc.VectorSubcoreMesh(core_axis_name="c", subcore_axis_name="s")


def _gate_body(x_ref, wg_ref, idx1_ref, idx2_ref, g1_ref, g2_ref, me_ref,
               xbf_ref):
    xv = x_ref[...]
    xbf_ref[...] = xv.astype(jnp.bfloat16)
    wgv = wg_ref[...]
    lg = jnp.dot(xv, wgv, preferred_element_type=jnp.float32)
    lane = lax.broadcasted_iota(jnp.int32, lg.shape, 1)
    valid = lane < E
    neg = jnp.float32(-1e30)
    lgm = jnp.where(valid, lg, neg)
    mx = jnp.max(lgm, axis=1, keepdims=True)
    ex = jnp.where(valid, jnp.exp(lgm - mx), 0.0)
    gates = ex / jnp.sum(ex, axis=1, keepdims=True)
    big = jnp.int32(1 << 20)
    i1 = jnp.min(jnp.where(lgm == mx, lane, big), axis=1, keepdims=True)
    lg2 = jnp.where(lane == i1, neg, lgm)
    mx2 = jnp.max(lg2, axis=1, keepdims=True)
    i2 = jnp.min(jnp.where(lg2 == mx2, lane, big), axis=1, keepdims=True)
    idx1_ref[...] = i1
    idx2_ref[...] = i2
    g1_ref[...] = jnp.sum(jnp.where(lane == i1, gates, 0.0), axis=1,
                          keepdims=True)
    g2_ref[...] = jnp.sum(jnp.where(lane == i2, gates, 0.0), axis=1,
                          keepdims=True)
    me_ref[...] = (jnp.sum(gates, axis=0, keepdims=True) / T)[:, :16]


_gate = pl.pallas_call(
    _gate_body,
    out_shape=[
        jax.ShapeDtypeStruct((T, 1), jnp.int32),
        jax.ShapeDtypeStruct((T, 1), jnp.int32),
        jax.ShapeDtypeStruct((T, 1), jnp.float32),
        jax.ShapeDtypeStruct((T, 1), jnp.float32),
        jax.ShapeDtypeStruct((1, 16), jnp.float32),
        jax.ShapeDtypeStruct((T, D), jnp.bfloat16),
    ],
)


@functools.partial(
    pl.kernel,
    mesh=_mesh,
    compiler_params=pltpu.CompilerParams(needs_layout_passes=False),
    out_type=[
        jax.ShapeDtypeStruct((E * C,), jnp.int32),
        jax.ShapeDtypeStruct((T,), jnp.int32),
        jax.ShapeDtypeStruct((T,), jnp.int32),
        jax.ShapeDtypeStruct((E * C,), jnp.float32),
        jax.ShapeDtypeStruct((16,), jnp.float32),
    ],
    scratch_types=[
        pltpu.VMEM((T,), jnp.int32),
        pltpu.VMEM((T,), jnp.int32),
        pltpu.VMEM((T,), jnp.float32),
        pltpu.VMEM((T,), jnp.float32),
        pltpu.VMEM((16,), jnp.float32),
        pltpu.VMEM((E * C,), jnp.int32),
        pltpu.VMEM((T,), jnp.int32),
        pltpu.VMEM((T,), jnp.int32),
        pltpu.VMEM((T,), jnp.float32),
        pltpu.VMEM((T,), jnp.float32),
        pltpu.VMEM((E * C,), jnp.float32),
        pltpu.VMEM((16,), jnp.float32),
    ],
)
def _route(idx1_h, idx2_h, g1_h, g2_h, me_h,
           src_h, slot1_h, slot2_h, wslot_h, laux_h,
           vidx1, vidx2, vg1, vg2, vme, vsrc, vslot1, vslot2, vgw1, vgw2,
           vwslot, vlaux):
    wid = lax.axis_index("s") * 2 + lax.axis_index("c")

    @pl.when(wid == 0)
    def _():
        pltpu.sync_copy(idx1_h, vidx1)
        pltpu.sync_copy(idx2_h, vidx2)
        pltpu.sync_copy(g1_h, vg1)
        pltpu.sync_copy(g2_h, vg2)
        pltpu.sync_copy(me_h, vme)
        iota16 = lax.iota(jnp.int32, 16)

        def zbody(i, c):
            vsrc[pl.ds(i * 16, 16)] = jnp.zeros((16,), jnp.int32)
            vwslot[pl.ds(i * 16, 16)] = jnp.zeros((16,), jnp.float32)
            return c

        lax.fori_loop(0, (E * C) // 16, zbody, 0)

        def make_pass(vidx, vg, vslot, vgw):
            def body(i, bases):
                ev = vidx[pl.ds(i * 16, 16)]
                gv = vg[pl.ds(i * 16, 16)]
                tvec = i * 16 + iota16
                locv = jnp.zeros((16,), jnp.int32)
                newb = []
                for e in range(E):
                    m = ev == e
                    ones = jnp.where(m, jnp.int32(1), jnp.int32(0))
                    pc = plsc.cumsum(ones)
                    locv = jnp.where(m, bases[e] + pc - 1, locv)
                    newb.append(bases[e] + jnp.sum(ones))
                kept = locv < C
                slotv = jnp.where(kept, ev * C + locv, 0)
                vslot[pl.ds(i * 16, 16)] = slotv
                vgw[pl.ds(i * 16, 16)] = jnp.where(kept, gv, jnp.float32(0.0))
                plsc.store_scatter(vsrc, [slotv], tvec, mask=kept)
                return tuple(newb)
            return body

        zero8 = (jnp.int32(0),) * E
        b1c = lax.fori_loop(0, T // 16, make_pass(vidx1, vg1, vslot1, vgw1),
                            zero8)
        cntv = jnp.zeros((16,), jnp.int32)
        for e in range(E):
            cntv = jnp.where(iota16 == e, b1c[e], cntv)
        s = jnp.sum(vme[...] * cntv.astype(jnp.float32))
        vlaux[...] = jnp.full((16,), jnp.float32(0.0), jnp.float32) + \
            s * jnp.float32(E / T)
        b2c = lax.fori_loop(0, T // 16, make_pass(vidx2, vg2, vslot2, vgw2),
                            b1c)
        dead = jnp.int32(0)
        for e in reversed(range(E)):
            dead = jnp.where(b2c[e] < C, e * C + b2c[e], dead)

        def nbody(i, c):
            a = vgw1[pl.ds(i * 16, 16)]
            b = vgw2[pl.ds(i * 16, 16)]
            den = jnp.maximum(a + b, jnp.float32(1e-9))
            g1n = a / den
            g2n = b / den
            k1 = a > 0
            k2 = b > 0
            s1 = jnp.where(k1, vslot1[pl.ds(i * 16, 16)], dead)
            s2 = jnp.where(k2, vslot2[pl.ds(i * 16, 16)], dead)
            vslot1[pl.ds(i * 16, 16)] = s1
            vslot2[pl.ds(i * 16, 16)] = s2
            plsc.store_scatter(vwslot, [s1], g1n, mask=k1)
            plsc.store_scatter(vwslot, [s2], g2n, mask=k2)
            return c

        lax.fori_loop(0, T // 16, nbody, 0)
        pltpu.sync_copy(vsrc, src_h)
        pltpu.sync_copy(vslot1, slot1_h)
        pltpu.sync_copy(vslot2, slot2_h)
        pltpu.sync_copy(vwslot, wslot_h)
        pltpu.sync_copy(vlaux, laux_h)


_SLOTS_PER_TILE = (E * C) // 32
_DCHUNK = 32
_DN = _SLOTS_PER_TILE // _DCHUNK


@functools.partial(
    pl.kernel,
    mesh=_mesh,
    compiler_params=pltpu.CompilerParams(needs_layout_passes=False),
    out_type=jax.ShapeDtypeStruct((E * C, D // 2), jnp.int32),
    scratch_types=[
        pltpu.VMEM((_SLOTS_PER_TILE,), jnp.int32),
        pltpu.VMEM((_DCHUNK, D // 2), jnp.int32),
        pltpu.VMEM((_DCHUNK, D // 2), jnp.int32),
        pltpu.SemaphoreType.DMA,
        pltpu.SemaphoreType.DMA,
        pltpu.SemaphoreType.DMA,
        pltpu.SemaphoreType.DMA,
    ],
)
def _dispatch(x_h, src_h, xe_h, idxv, rows0, rows1, sg0, sg1, so0, so1):
    wid = lax.axis_index("s") * 2 + lax.axis_index("c")
    base = wid * _SLOTS_PER_TILE
    pltpu.sync_copy(src_h.at[pl.ds(base, _SLOTS_PER_TILE)], idxv)
    rows = [rows0, rows1]
    sg = [sg0, sg1]
    so = [so0, so1]

    def start_gather(ch):
        return pltpu.async_copy(
            x_h.at[idxv.at[pl.ds(ch * _DCHUNK, _DCHUNK)]],
            rows[ch % 2], sg[ch % 2])

    gh = [start_gather(0), start_gather(1)]
    sh = [None] * _DN
    for ch in range(_DN):
        gh[ch % 2].wait()
        sh[ch] = pltpu.async_copy(
            rows[ch % 2], xe_h.at[pl.ds(base + ch * _DCHUNK, _DCHUNK)],
            so[ch % 2])
        if ch + 2 < _DN:
            sh[ch].wait()
            gh[ch % 2] = start_gather(ch + 2)
    for ch in (_DN - 2, _DN - 1):
        sh[ch].wait()


def _ffn_body(xe_ref, w1_ref, b1_ref, w2_ref, b2_ref, ws_ref, eo_ref):
    xv = xe_ref[...]
    w1b = w1_ref[0].astype(jnp.bfloat16)
    h = jnp.dot(xv, w1b, preferred_element_type=jnp.float32) + \
        b1_ref[0]
    h = jnp.maximum(h, 0.0)
    o = jnp.dot(h, w2_ref[0], preferred_element_type=jnp.float32) + \
        b2_ref[0]
    eo_ref[...] = o * ws_ref[...]


_ffn = pl.pallas_call(
    _ffn_body,
    grid=(E,),
    in_specs=[
        pl.BlockSpec((C, D), lambda e: (e, 0)),
        pl.BlockSpec((1, D, F), lambda e: (e, 0, 0)),
        pl.BlockSpec((1, 1, F), lambda e: (e, 0, 0)),
        pl.BlockSpec((1, F, D), lambda e: (e, 0, 0)),
        pl.BlockSpec((1, 1, D), lambda e: (e, 0, 0)),
        pl.BlockSpec((C, 1), lambda e: (e, 0)),
    ],
    out_specs=pl.BlockSpec((C, D), lambda e: (e, 0)),
    out_shape=jax.ShapeDtypeStruct((E * C, D), jnp.float32),
    compiler_params=pltpu.CompilerParams(
        dimension_semantics=("arbitrary",)),
)


_TOK_PER_TILE = T // 32
_CCHUNK = 16
_CN = _TOK_PER_TILE // _CCHUNK


@functools.partial(
    pl.kernel,
    mesh=_mesh,
    compiler_params=pltpu.CompilerParams(needs_layout_passes=False),
    out_type=jax.ShapeDtypeStruct((T, D), jnp.float32),
    scratch_types=[
        pltpu.VMEM((_TOK_PER_TILE,), jnp.int32),
        pltpu.VMEM((_TOK_PER_TILE,), jnp.int32),
        pltpu.VMEM((_CCHUNK, D), jnp.float32),
        pltpu.VMEM((_CCHUNK, D), jnp.float32),
        pltpu.VMEM((_CCHUNK, D), jnp.float32),
        pltpu.VMEM((_CCHUNK, D), jnp.float32),
        pltpu.SemaphoreType.DMA,
        pltpu.SemaphoreType.DMA,
        pltpu.SemaphoreType.DMA,
        pltpu.SemaphoreType.DMA,
    ],
)
def _combine(eo_h, slot1_h, slot2_h, out_h,
             s1v, s2v, r1a, r1b, r2a, r2b, sg0, sg1, so0, so1):
    wid = lax.axis_index("s") * 2 + lax.axis_index("c")
    base = wid * _TOK_PER_TILE
    pltpu.sync_copy(slot1_h.at[pl.ds(base, _TOK_PER_TILE)], s1v)
    pltpu.sync_copy(slot2_h.at[pl.ds(base, _TOK_PER_TILE)], s2v)
    r1 = [r1a, r1b]
    r2 = [r2a, r2b]
    sg = [sg0, sg1]
    so = [so0, so1]

    def start_gather(ch):
        p = ch % 2
        h1 = pltpu.async_copy(
            eo_h.at[s1v.at[pl.ds(ch * _CCHUNK, _CCHUNK)]], r1[p], sg[p])
        h2 = pltpu.async_copy(
            eo_h.at[s2v.at[pl.ds(ch * _CCHUNK, _CCHUNK)]], r2[p], sg[p])
        return (h1, h2)

    gh = [start_gather(0), start_gather(1)]
    sh = [None] * _CN
    for ch in range(_CN):
        p = ch % 2
        gh[p][0].wait()
        gh[p][1].wait()

        for j in range(_CCHUNK):
            def cbj(k, c, p=p, j=j):
                for u in range(4):
                    sl = pl.ds(k * 64 + u * 16, 16)
                    plsc.addupdate(r1[p].at[j, sl], r2[p][j, sl])
                return c
            lax.fori_loop(0, D // 64, cbj, 0)
        sh[ch] = pltpu.async_copy(
            r1[p], out_h.at[pl.ds(base + ch * _CCHUNK, _CCHUNK)], so[p])
        if ch + 2 < _CN:
            sh[ch].wait()
            gh[p] = start_gather(ch + 2)
    sh[_CN - 2].wait()
    sh[_CN - 1].wait()


def kernel(x, wg, w1, b1, w2, b2):
    xt = x.reshape(T, D)
    wgp = jnp.pad(wg, ((0, 0), (0, 128 - E)))
    i1, i2, g1r, g2r, me, xbf = _gate(xt, wgp)
    src, slot1, slot2, wslot, laux = _route(
        i1.reshape(T), i2.reshape(T), g1r.reshape(T), g2r.reshape(T),
        me.reshape(16))
    xbi = lax.bitcast_convert_type(
        xbf.reshape(T, D // 2, 2), jnp.int32)
    xei = _dispatch(xbi, src)
    xe = lax.bitcast_convert_type(xei, jnp.bfloat16).reshape(E * C, D)
    eo = _ffn(xe, w1, b1.reshape(E, 1, F), w2, b2.reshape(E, 1, D),
              wslot.reshape(E * C, 1))
    out = _combine(eo, slot1, slot2)
    return out.reshape(x.shape), laux[0]


# speedup vs baseline: 1.5820x; 1.5820x over previous
import functools

import jax
import jax.numpy as jnp
from jax import lax
from jax.experimental import pallas as pl
from jax.experimental.pallas import tpu as pltpu
from jax.experimental.pallas import tpu_sc as plsc

T = 2048
D = 1024
E = 8
F = 2048
C = (2 * T) // E

_mesh = plsc.VectorSubcoreMesh(core_axis_name="c", subcore_axis_name="s")


def _gate_body(x_ref, wg_ref, idx1_ref, idx2_ref, g1_ref, g2_ref, me_ref):
    xv = x_ref[...]
    wgv = wg_ref[...]
    lg = jnp.dot(xv, wgv, preferred_element_type=jnp.float32)
    lane = lax.broadcasted_iota(jnp.int32, lg.shape, 1)
    valid = lane < E
    neg = jnp.float32(-1e30)
    lgm = jnp.where(valid, lg, neg)
    mx = jnp.max(lgm, axis=1, keepdims=True)
    ex = jnp.where(valid, jnp.exp(lgm - mx), 0.0)
    gates = ex / jnp.sum(ex, axis=1, keepdims=True)
    big = jnp.int32(1 << 20)
    i1 = jnp.min(jnp.where(lgm == mx, lane, big), axis=1, keepdims=True)
    lg2 = jnp.where(lane == i1, neg, lgm)
    mx2 = jnp.max(lg2, axis=1, keepdims=True)
    i2 = jnp.min(jnp.where(lg2 == mx2, lane, big), axis=1, keepdims=True)
    idx1_ref[...] = i1
    idx2_ref[...] = i2
    g1_ref[...] = jnp.sum(jnp.where(lane == i1, gates, 0.0), axis=1,
                          keepdims=True)
    g2_ref[...] = jnp.sum(jnp.where(lane == i2, gates, 0.0), axis=1,
                          keepdims=True)
    me_ref[...] = (jnp.sum(gates, axis=0, keepdims=True) / T)[:, :16]


_gate = pl.pallas_call(
    _gate_body,
    out_shape=[
        jax.ShapeDtypeStruct((T, 1), jnp.int32),
        jax.ShapeDtypeStruct((T, 1), jnp.int32),
        jax.ShapeDtypeStruct((T, 1), jnp.float32),
        jax.ShapeDtypeStruct((T, 1), jnp.float32),
        jax.ShapeDtypeStruct((1, 16), jnp.float32),
    ],
)


@functools.partial(
    pl.kernel,
    mesh=_mesh,
    compiler_params=pltpu.CompilerParams(needs_layout_passes=False),
    out_type=[
        jax.ShapeDtypeStruct((E * C,), jnp.int32),
        jax.ShapeDtypeStruct((T,), jnp.int32),
        jax.ShapeDtypeStruct((T,), jnp.int32),
        jax.ShapeDtypeStruct((E * C,), jnp.float32),
        jax.ShapeDtypeStruct((16,), jnp.float32),
    ],
    scratch_types=[
        pltpu.VMEM((T,), jnp.int32),
        pltpu.VMEM((T,), jnp.int32),
        pltpu.VMEM((T,), jnp.float32),
        pltpu.VMEM((T,), jnp.float32),
        pltpu.VMEM((16,), jnp.float32),
        pltpu.VMEM((E * C,), jnp.int32),
        pltpu.VMEM((T,), jnp.int32),
        pltpu.VMEM((T,), jnp.int32),
        pltpu.VMEM((T,), jnp.float32),
        pltpu.VMEM((T,), jnp.float32),
        pltpu.VMEM((E * C,), jnp.float32),
        pltpu.VMEM((16,), jnp.float32),
    ],
)
def _route(idx1_h, idx2_h, g1_h, g2_h, me_h,
           src_h, slot1_h, slot2_h, wslot_h, laux_h,
           vidx1, vidx2, vg1, vg2, vme, vsrc, vslot1, vslot2, vgw1, vgw2,
           vwslot, vlaux):
    wid = lax.axis_index("s") * 2 + lax.axis_index("c")

    @pl.when(wid == 0)
    def _():
        pltpu.sync_copy(idx1_h, vidx1)
        pltpu.sync_copy(idx2_h, vidx2)
        pltpu.sync_copy(g1_h, vg1)
        pltpu.sync_copy(g2_h, vg2)
        pltpu.sync_copy(me_h, vme)
        iota16 = lax.iota(jnp.int32, 16)

        def zbody(i, c):
            vsrc[pl.ds(i * 16, 16)] = jnp.zeros((16,), jnp.int32)
            vwslot[pl.ds(i * 16, 16)] = jnp.zeros((16,), jnp.float32)
            return c

        lax.fori_loop(0, (E * C) // 16, zbody, 0)

        def make_pass(vidx, vg, vslot, vgw):
            def body(i, bases):
                ev = vidx[pl.ds(i * 16, 16)]
                gv = vg[pl.ds(i * 16, 16)]
                tvec = i * 16 + iota16
                locv = jnp.zeros((16,), jnp.int32)
                newb = []
                for e in range(E):
                    m = ev == e
                    ones = jnp.where(m, jnp.int32(1), jnp.int32(0))
                    pc = plsc.cumsum(ones)
                    locv = jnp.where(m, bases[e] + pc - 1, locv)
                    newb.append(bases[e] + jnp.sum(ones))
                kept = locv < C
                slotv = jnp.where(kept, ev * C + locv, 0)
                vslot[pl.ds(i * 16, 16)] = slotv
                vgw[pl.ds(i * 16, 16)] = jnp.where(kept, gv, jnp.float32(0.0))
                plsc.store_scatter(vsrc, [slotv], tvec, mask=kept)
                return tuple(newb)
            return body

        zero8 = (jnp.int32(0),) * E
        b1c = lax.fori_loop(0, T // 16, make_pass(vidx1, vg1, vslot1, vgw1),
                            zero8)
        cntv = jnp.zeros((16,), jnp.int32)
        for e in range(E):
            cntv = jnp.where(iota16 == e, b1c[e], cntv)
        s = jnp.sum(vme[...] * cntv.astype(jnp.float32))
        vlaux[...] = jnp.full((16,), jnp.float32(0.0), jnp.float32) + \
            s * jnp.float32(E / T)
        b2c = lax.fori_loop(0, T // 16, make_pass(vidx2, vg2, vslot2, vgw2),
                            b1c)
        dead = jnp.int32(0)
        for e in reversed(range(E)):
            dead = jnp.where(b2c[e] < C, e * C + b2c[e], dead)

        def nbody(i, c):
            a = vgw1[pl.ds(i * 16, 16)]
            b = vgw2[pl.ds(i * 16, 16)]
            den = jnp.maximum(a + b, jnp.float32(1e-9))
            g1n = a / den
            g2n = b / den
            k1 = a > 0
            k2 = b > 0
            s1 = jnp.where(k1, vslot1[pl.ds(i * 16, 16)], dead)
            s2 = jnp.where(k2, vslot2[pl.ds(i * 16, 16)], dead)
            vslot1[pl.ds(i * 16, 16)] = s1
            vslot2[pl.ds(i * 16, 16)] = s2
            plsc.store_scatter(vwslot, [s1], g1n, mask=k1)
            plsc.store_scatter(vwslot, [s2], g2n, mask=k2)
            return c

        lax.fori_loop(0, T // 16, nbody, 0)
        pltpu.sync_copy(vsrc, src_h)
        pltpu.sync_copy(vslot1, slot1_h)
        pltpu.sync_copy(vslot2, slot2_h)
        pltpu.sync_copy(vwslot, wslot_h)
        pltpu.sync_copy(vlaux, laux_h)


_SLOTS_PER_TILE = (E * C) // 32
_DCHUNK = 32
_DN = _SLOTS_PER_TILE // _DCHUNK


@functools.partial(
    pl.kernel,
    mesh=_mesh,
    compiler_params=pltpu.CompilerParams(needs_layout_passes=False),
    out_type=jax.ShapeDtypeStruct((E * C, D), jnp.float32),
    scratch_types=[
        pltpu.VMEM((_SLOTS_PER_TILE,), jnp.int32),
        pltpu.VMEM((_DCHUNK, D), jnp.float32),
        pltpu.VMEM((_DCHUNK, D), jnp.float32),
        pltpu.SemaphoreType.DMA,
        pltpu.SemaphoreType.DMA,
        pltpu.SemaphoreType.DMA,
        pltpu.SemaphoreType.DMA,
    ],
)
def _dispatch(x_h, src_h, xe_h, idxv, rows0, rows1, sg0, sg1, so0, so1):
    wid = lax.axis_index("s") * 2 + lax.axis_index("c")
    base = wid * _SLOTS_PER_TILE
    pltpu.sync_copy(src_h.at[pl.ds(base, _SLOTS_PER_TILE)], idxv)
    rows = [rows0, rows1]
    sg = [sg0, sg1]
    so = [so0, so1]

    def start_gather(ch):
        return pltpu.async_copy(
            x_h.at[idxv.at[pl.ds(ch * _DCHUNK, _DCHUNK)]],
            rows[ch % 2], sg[ch % 2])

    gh = [start_gather(0), start_gather(1)]
    sh = [None] * _DN
    for ch in range(_DN):
        gh[ch % 2].wait()
        sh[ch] = pltpu.async_copy(
            rows[ch % 2], xe_h.at[pl.ds(base + ch * _DCHUNK, _DCHUNK)],
            so[ch % 2])
        if ch + 2 < _DN:
            sh[ch].wait()
            gh[ch % 2] = start_gather(ch + 2)
    for ch in (_DN - 2, _DN - 1):
        sh[ch].wait()


def _ffn_body(xe_ref, w1_ref, b1_ref, w2_ref, b2_ref, ws_ref, eo_ref):
    xv = xe_ref[...]
    h = jnp.dot(xv, w1_ref[0], preferred_element_type=jnp.float32) + \
        b1_ref[0]
    h = jnp.maximum(h, 0.0)
    o = jnp.dot(h, w2_ref[0], preferred_element_type=jnp.float32) + \
        b2_ref[0]
    eo_ref[...] = o * ws_ref[...]


_ffn = pl.pallas_call(
    _ffn_body,
    grid=(E,),
    in_specs=[
        pl.BlockSpec((C, D), lambda e: (e, 0)),
        pl.BlockSpec((1, D, F), lambda e: (e, 0, 0)),
        pl.BlockSpec((1, 1, F), lambda e: (e, 0, 0)),
        pl.BlockSpec((1, F, D), lambda e: (e, 0, 0)),
        pl.BlockSpec((1, 1, D), lambda e: (e, 0, 0)),
        pl.BlockSpec((C, 1), lambda e: (e, 0)),
    ],
    out_specs=pl.BlockSpec((C, D), lambda e: (e, 0)),
    out_shape=jax.ShapeDtypeStruct((E * C, D), jnp.float32),
    compiler_params=pltpu.CompilerParams(
        dimension_semantics=("arbitrary",)),
)


_TOK_PER_TILE = T // 32
_CCHUNK = 32


@functools.partial(
    pl.kernel,
    mesh=_mesh,
    compiler_params=pltpu.CompilerParams(needs_layout_passes=False),
    out_type=jax.ShapeDtypeStruct((T, D), jnp.float32),
    scratch_types=[
        pltpu.VMEM((_TOK_PER_TILE,), jnp.int32),
        pltpu.VMEM((_TOK_PER_TILE,), jnp.int32),
        pltpu.VMEM((_CCHUNK, D), jnp.float32),
        pltpu.VMEM((_CCHUNK, D), jnp.float32),
        pltpu.VMEM((_CCHUNK, D), jnp.float32),
        pltpu.SemaphoreType.DMA,
        pltpu.SemaphoreType.DMA,
        pltpu.SemaphoreType.DMA,
        pltpu.SemaphoreType.DMA,
        pltpu.SemaphoreType.DMA,
    ],
)
def _combine(eo_h, slot1_h, slot2_h, out_h,
             s1v, s2v, r1a, r1b, r2, sg0, sg1, sg2, so0, so1):
    wid = lax.axis_index("s") * 2 + lax.axis_index("c")
    base = wid * _TOK_PER_TILE
    pltpu.sync_copy(slot1_h.at[pl.ds(base, _TOK_PER_TILE)], s1v)
    pltpu.sync_copy(slot2_h.at[pl.ds(base, _TOK_PER_TILE)], s2v)

    h1a = pltpu.async_copy(eo_h.at[s1v.at[pl.ds(0, _CCHUNK)]], r1a, sg0)
    h2a = pltpu.async_copy(eo_h.at[s2v.at[pl.ds(0, _CCHUNK)]], r2, sg1)
    h1b = pltpu.async_copy(eo_h.at[s1v.at[pl.ds(_CCHUNK, _CCHUNK)]], r1b, sg2)

    def add_rows(dst, src):
        for j in range(_CCHUNK):
            def cbj(k, c, j=j):
                for u in range(4):
                    sl = pl.ds(k * 64 + u * 16, 16)
                    plsc.addupdate(dst.at[j, sl], src[j, sl])
                return c
            lax.fori_loop(0, D // 64, cbj, 0)

    h1a.wait()
    h2a.wait()
    add_rows(r1a, r2)
    sha = pltpu.async_copy(r1a, out_h.at[pl.ds(base, _CCHUNK)], so0)
    h2b = pltpu.async_copy(eo_h.at[s2v.at[pl.ds(_CCHUNK, _CCHUNK)]], r2, sg1)
    h1b.wait()
    h2b.wait()
    add_rows(r1b, r2)
    shb = pltpu.async_copy(
        r1b, out_h.at[pl.ds(base + _CCHUNK, _CCHUNK)], so1)
    sha.wait()
    shb.wait()


def kernel(x, wg, w1, b1, w2, b2):
    xt = x.reshape(T, D)
    wgp = jnp.pad(wg, ((0, 0), (0, 128 - E)))
    i1, i2, g1r, g2r, me = _gate(xt, wgp)
    src, slot1, slot2, wslot, laux = _route(
        i1.reshape(T), i2.reshape(T), g1r.reshape(T), g2r.reshape(T),
        me.reshape(16))
    xe = _dispatch(xt, src)
    eo = _ffn(xe, w1, b1.reshape(E, 1, F), w2, b2.reshape(E, 1, D),
              wslot.reshape(E * C, 1))
    out = _combine(eo, slot1, slot2)
    return out.reshape(x.shape), laux[0]


# speedup vs baseline: 1.6448x; 1.0397x over previous
import functools

import jax
import jax.numpy as jnp
from jax import lax
from jax.experimental import pallas as pl
from jax.experimental.pallas import tpu as pltpu
from jax.experimental.pallas import tpu_sc as plsc

T = 2048
D = 1024
E = 8
F = 2048
C = (2 * T) // E

_mesh = plsc.VectorSubcoreMesh(core_axis_name="c", subcore_axis_name="s")


def _gate_body(x_ref, wg_ref, idx1_ref, idx2_ref, g1_ref, g2_ref, me_ref):
    xv = x_ref[...]
    wgv = wg_ref[...]
    lg = jnp.dot(xv, wgv, preferred_element_type=jnp.float32)
    lane = lax.broadcasted_iota(jnp.int32, lg.shape, 1)
    valid = lane < E
    neg = jnp.float32(-1e30)
    lgm = jnp.where(valid, lg, neg)
    mx = jnp.max(lgm, axis=1, keepdims=True)
    ex = jnp.where(valid, jnp.exp(lgm - mx), 0.0)
    gates = ex / jnp.sum(ex, axis=1, keepdims=True)
    big = jnp.int32(1 << 20)
    i1 = jnp.min(jnp.where(lgm == mx, lane, big), axis=1, keepdims=True)
    lg2 = jnp.where(lane == i1, neg, lgm)
    mx2 = jnp.max(lg2, axis=1, keepdims=True)
    i2 = jnp.min(jnp.where(lg2 == mx2, lane, big), axis=1, keepdims=True)
    idx1_ref[...] = i1
    idx2_ref[...] = i2
    g1_ref[...] = jnp.sum(jnp.where(lane == i1, gates, 0.0), axis=1,
                          keepdims=True)
    g2_ref[...] = jnp.sum(jnp.where(lane == i2, gates, 0.0), axis=1,
                          keepdims=True)
    me_ref[...] = (jnp.sum(gates, axis=0, keepdims=True) / T)[:, :16]


_gate = pl.pallas_call(
    _gate_body,
    out_shape=[
        jax.ShapeDtypeStruct((T, 1), jnp.int32),
        jax.ShapeDtypeStruct((T, 1), jnp.int32),
        jax.ShapeDtypeStruct((T, 1), jnp.float32),
        jax.ShapeDtypeStruct((T, 1), jnp.float32),
        jax.ShapeDtypeStruct((1, 16), jnp.float32),
    ],
)


@functools.partial(
    pl.kernel,
    mesh=_mesh,
    compiler_params=pltpu.CompilerParams(needs_layout_passes=False),
    out_type=[
        jax.ShapeDtypeStruct((E * C,), jnp.int32),
        jax.ShapeDtypeStruct((T,), jnp.int32),
        jax.ShapeDtypeStruct((T,), jnp.int32),
        jax.ShapeDtypeStruct((E * C,), jnp.float32),
        jax.ShapeDtypeStruct((16,), jnp.float32),
    ],
    scratch_types=[
        pltpu.VMEM((T,), jnp.int32),
        pltpu.VMEM((T,), jnp.int32),
        pltpu.VMEM((T,), jnp.float32),
        pltpu.VMEM((T,), jnp.float32),
        pltpu.VMEM((16,), jnp.float32),
        pltpu.VMEM((E * C,), jnp.int32),
        pltpu.VMEM((T,), jnp.int32),
        pltpu.VMEM((T,), jnp.int32),
        pltpu.VMEM((T,), jnp.float32),
        pltpu.VMEM((T,), jnp.float32),
        pltpu.VMEM((E * C,), jnp.float32),
        pltpu.VMEM((16,), jnp.float32),
    ],
)
def _route(idx1_h, idx2_h, g1_h, g2_h, me_h,
           src_h, slot1_h, slot2_h, wslot_h, laux_h,
           vidx1, vidx2, vg1, vg2, vme, vsrc, vslot1, vslot2, vgw1, vgw2,
           vwslot, vlaux):
    wid = lax.axis_index("s") * 2 + lax.axis_index("c")

    @pl.when(wid == 0)
    def _():
        pltpu.sync_copy(idx1_h, vidx1)
        pltpu.sync_copy(idx2_h, vidx2)
        pltpu.sync_copy(g1_h, vg1)
        pltpu.sync_copy(g2_h, vg2)
        pltpu.sync_copy(me_h, vme)
        iota16 = lax.iota(jnp.int32, 16)

        def zbody(i, c):
            vsrc[pl.ds(i * 16, 16)] = jnp.zeros((16,), jnp.int32)
            vwslot[pl.ds(i * 16, 16)] = jnp.zeros((16,), jnp.float32)
            return c

        lax.fori_loop(0, (E * C) // 16, zbody, 0)

        def make_pass(vidx, vg, vslot, vgw):
            def body(i, bases):
                ev = vidx[pl.ds(i * 16, 16)]
                gv = vg[pl.ds(i * 16, 16)]
                tvec = i * 16 + iota16
                is_lo = ev < 4
                sh_lo = ev * 8
                sh_hi = (ev - 4) * 8
                one = jnp.int32(1)
                packed_lo = jnp.where(is_lo, one << sh_lo, 0)
                packed_hi = jnp.where(is_lo, 0, one << sh_hi)
                pc_lo = plsc.cumsum(packed_lo)
                pc_hi = plsc.cumsum(packed_hi)
                rank = jnp.where(is_lo, pc_lo >> sh_lo, pc_hi >> sh_hi) & 255
                basev = jnp.zeros((16,), jnp.int32)
                for e in range(E):
                    basev = jnp.where(ev == e, bases[e], basev)
                locv = basev + rank - 1
                tl = pc_lo[15]
                th = pc_hi[15]
                newb = [bases[e] + ((tl >> (8 * e)) & 255) for e in range(4)]
                newb += [bases[4 + e] + ((th >> (8 * e)) & 255)
                         for e in range(4)]
                kept = locv < C
                slotv = jnp.where(kept, ev * C + locv, 0)
                vslot[pl.ds(i * 16, 16)] = slotv
                vgw[pl.ds(i * 16, 16)] = jnp.where(kept, gv, jnp.float32(0.0))
                plsc.store_scatter(vsrc, [slotv], tvec, mask=kept)
                return tuple(newb)
            return body

        zero8 = (jnp.int32(0),) * E
        b1c = lax.fori_loop(0, T // 16, make_pass(vidx1, vg1, vslot1, vgw1),
                            zero8)
        cntv = jnp.zeros((16,), jnp.int32)
        for e in range(E):
            cntv = jnp.where(iota16 == e, b1c[e], cntv)
        s = jnp.sum(vme[...] * cntv.astype(jnp.float32))
        vlaux[...] = jnp.full((16,), jnp.float32(0.0), jnp.float32) + \
            s * jnp.float32(E / T)
        b2c = lax.fori_loop(0, T // 16, make_pass(vidx2, vg2, vslot2, vgw2),
                            b1c)
        dead = jnp.int32(0)
        for e in reversed(range(E)):
            dead = jnp.where(b2c[e] < C, e * C + b2c[e], dead)

        def nbody(i, c):
            a = vgw1[pl.ds(i * 16, 16)]
            b = vgw2[pl.ds(i * 16, 16)]
            den = jnp.maximum(a + b, jnp.float32(1e-9))
            g1n = a / den
            g2n = b / den
            k1 = a > 0
            k2 = b > 0
            s1 = jnp.where(k1, vslot1[pl.ds(i * 16, 16)], dead)
            s2 = jnp.where(k2, vslot2[pl.ds(i * 16, 16)], dead)
            vslot1[pl.ds(i * 16, 16)] = s1
            vslot2[pl.ds(i * 16, 16)] = s2
            plsc.store_scatter(vwslot, [s1], g1n, mask=k1)
            plsc.store_scatter(vwslot, [s2], g2n, mask=k2)
            return c

        lax.fori_loop(0, T // 16, nbody, 0)
        pltpu.sync_copy(vsrc, src_h)
        pltpu.sync_copy(vslot1, slot1_h)
        pltpu.sync_copy(vslot2, slot2_h)
        pltpu.sync_copy(vwslot, wslot_h)
        pltpu.sync_copy(vlaux, laux_h)


_SLOTS_PER_TILE = (E * C) // 32
_DCHUNK = 32
_DN = _SLOTS_PER_TILE // _DCHUNK


@functools.partial(
    pl.kernel,
    mesh=_mesh,
    compiler_params=pltpu.CompilerParams(needs_layout_passes=False),
    out_type=jax.ShapeDtypeStruct((E * C, D), jnp.float32),
    scratch_types=[
        pltpu.VMEM((_SLOTS_PER_TILE,), jnp.int32),
        pltpu.VMEM((_DCHUNK, D), jnp.float32),
        pltpu.VMEM((_DCHUNK, D), jnp.float32),
        pltpu.SemaphoreType.DMA,
        pltpu.SemaphoreType.DMA,
        pltpu.SemaphoreType.DMA,
        pltpu.SemaphoreType.DMA,
    ],
)
def _dispatch(x_h, src_h, xe_h, idxv, rows0, rows1, sg0, sg1, so0, so1):
    wid = lax.axis_index("s") * 2 + lax.axis_index("c")
    base = wid * _SLOTS_PER_TILE
    pltpu.sync_copy(src_h.at[pl.ds(base, _SLOTS_PER_TILE)], idxv)
    rows = [rows0, rows1]
    sg = [sg0, sg1]
    so = [so0, so1]

    def start_gather(ch):
        return pltpu.async_copy(
            x_h.at[idxv.at[pl.ds(ch * _DCHUNK, _DCHUNK)]],
            rows[ch % 2], sg[ch % 2])

    gh = [start_gather(0), start_gather(1)]
    sh = [None] * _DN
    for ch in range(_DN):
        gh[ch % 2].wait()
        sh[ch] = pltpu.async_copy(
            rows[ch % 2], xe_h.at[pl.ds(base + ch * _DCHUNK, _DCHUNK)],
            so[ch % 2])
        if ch + 2 < _DN:
            sh[ch].wait()
            gh[ch % 2] = start_gather(ch + 2)
    for ch in (_DN - 2, _DN - 1):
        sh[ch].wait()


def _ffn_body(xe_ref, w1_ref, b1_ref, w2_ref, b2_ref, ws_ref, eo_ref):
    xv = xe_ref[...]
    h = jnp.dot(xv, w1_ref[0], preferred_element_type=jnp.float32) + \
        b1_ref[0]
    h = jnp.maximum(h, 0.0)
    o = jnp.dot(h, w2_ref[0], preferred_element_type=jnp.float32) + \
        b2_ref[0]
    eo_ref[...] = o * ws_ref[...]


_ffn = pl.pallas_call(
    _ffn_body,
    grid=(E,),
    in_specs=[
        pl.BlockSpec((C, D), lambda e: (e, 0)),
        pl.BlockSpec((1, D, F), lambda e: (e, 0, 0)),
        pl.BlockSpec((1, 1, F), lambda e: (e, 0, 0)),
        pl.BlockSpec((1, F, D), lambda e: (e, 0, 0)),
        pl.BlockSpec((1, 1, D), lambda e: (e, 0, 0)),
        pl.BlockSpec((C, 1), lambda e: (e, 0)),
    ],
    out_specs=pl.BlockSpec((C, D), lambda e: (e, 0)),
    out_shape=jax.ShapeDtypeStruct((E * C, D), jnp.float32),
    compiler_params=pltpu.CompilerParams(
        dimension_semantics=("arbitrary",)),
)


_TOK_PER_TILE = T // 32
_CCHUNK = 16
_CN = _TOK_PER_TILE // _CCHUNK


@functools.partial(
    pl.kernel,
    mesh=_mesh,
    compiler_params=pltpu.CompilerParams(needs_layout_passes=False),
    out_type=jax.ShapeDtypeStruct((T, D), jnp.float32),
    scratch_types=[
        pltpu.VMEM((_TOK_PER_TILE,), jnp.int32),
        pltpu.VMEM((_TOK_PER_TILE,), jnp.int32),
        pltpu.VMEM((_CCHUNK, D), jnp.float32),
        pltpu.VMEM((_CCHUNK, D), jnp.float32),
        pltpu.VMEM((_CCHUNK, D), jnp.float32),
        pltpu.VMEM((_CCHUNK, D), jnp.float32),
        pltpu.SemaphoreType.DMA,
        pltpu.SemaphoreType.DMA,
        pltpu.SemaphoreType.DMA,
        pltpu.SemaphoreType.DMA,
    ],
)
def _combine(eo_h, slot1_h, slot2_h, out_h,
             s1v, s2v, r1a, r1b, r2a, r2b, sg0, sg1, so0, so1):
    wid = lax.axis_index("s") * 2 + lax.axis_index("c")
    base = wid * _TOK_PER_TILE
    pltpu.sync_copy(slot1_h.at[pl.ds(base, _TOK_PER_TILE)], s1v)
    pltpu.sync_copy(slot2_h.at[pl.ds(base, _TOK_PER_TILE)], s2v)
    r1 = [r1a, r1b]
    r2 = [r2a, r2b]
    sg = [sg0, sg1]
    so = [so0, so1]

    def start_gather(ch):
        p = ch % 2
        h1 = pltpu.async_copy(
            eo_h.at[s1v.at[pl.ds(ch * _CCHUNK, _CCHUNK)]], r1[p], sg[p])
        h2 = pltpu.async_copy(
            eo_h.at[s2v.at[pl.ds(ch * _CCHUNK, _CCHUNK)]], r2[p], sg[p])
        return (h1, h2)

    gh = [start_gather(0), start_gather(1)]
    sh = [None] * _CN
    for ch in range(_CN):
        p = ch % 2
        gh[p][0].wait()
        gh[p][1].wait()

        for j in range(_CCHUNK):
            def cbj(k, c, p=p, j=j):
                for u in range(4):
                    sl = pl.ds(k * 64 + u * 16, 16)
                    plsc.addupdate(r1[p].at[j, sl], r2[p][j, sl])
                return c
            lax.fori_loop(0, D // 64, cbj, 0)
        sh[ch] = pltpu.async_copy(
            r1[p], out_h.at[pl.ds(base + ch * _CCHUNK, _CCHUNK)], so[p])
        if ch + 2 < _CN:
            sh[ch].wait()
            gh[p] = start_gather(ch + 2)
    sh[_CN - 2].wait()
    sh[_CN - 1].wait()


def kernel(x, wg, w1, b1, w2, b2):
    xt = x.reshape(T, D)
    wgp = jnp.pad(wg, ((0, 0), (0, 128 - E)))
    i1, i2, g1r, g2r, me = _gate(xt, wgp)
    src, slot1, slot2, wslot, laux = _route(
        i1.reshape(T), i2.reshape(T), g1r.reshape(T), g2r.reshape(T),
        me.reshape(16))
    xe = _dispatch(xt, src)
    eo = _ffn(xe, w1, b1.reshape(E, 1, F), w2, b2.reshape(E, 1, D),
              wslot.reshape(E * C, 1))
    out = _combine(eo, slot1, slot2)
    return out.reshape(x.shape), laux[0]


# speedup vs baseline: 1.6658x; 1.0128x over previous
import functools

import jax
import jax.numpy as jnp
from jax import lax
from jax.experimental import pallas as pl
from jax.experimental.pallas import tpu as pltpu
from jax.experimental.pallas import tpu_sc as plsc

T = 2048
D = 1024
E = 8
F = 2048
C = (2 * T) // E

_mesh = plsc.VectorSubcoreMesh(core_axis_name="c", subcore_axis_name="s")


def _gate_body(x_ref, wg_ref, idx1_ref, idx2_ref, g1_ref, g2_ref, me_ref):
    xv = x_ref[...]
    wgv = wg_ref[...]
    lg = jnp.dot(xv, wgv, preferred_element_type=jnp.float32)
    lane = lax.broadcasted_iota(jnp.int32, lg.shape, 1)
    valid = lane < E
    neg = jnp.float32(-1e30)
    lgm = jnp.where(valid, lg, neg)
    mx = jnp.max(lgm, axis=1, keepdims=True)
    ex = jnp.where(valid, jnp.exp(lgm - mx), 0.0)
    gates = ex / jnp.sum(ex, axis=1, keepdims=True)
    big = jnp.int32(1 << 20)
    i1 = jnp.min(jnp.where(lgm == mx, lane, big), axis=1, keepdims=True)
    lg2 = jnp.where(lane == i1, neg, lgm)
    mx2 = jnp.max(lg2, axis=1, keepdims=True)
    i2 = jnp.min(jnp.where(lg2 == mx2, lane, big), axis=1, keepdims=True)
    idx1_ref[...] = i1
    idx2_ref[...] = i2
    g1_ref[...] = jnp.sum(jnp.where(lane == i1, gates, 0.0), axis=1,
                          keepdims=True)
    g2_ref[...] = jnp.sum(jnp.where(lane == i2, gates, 0.0), axis=1,
                          keepdims=True)
    me_ref[...] = (jnp.sum(gates, axis=0, keepdims=True) / T)[:, :16]


_gate = pl.pallas_call(
    _gate_body,
    out_shape=[
        jax.ShapeDtypeStruct((T, 1), jnp.int32),
        jax.ShapeDtypeStruct((T, 1), jnp.int32),
        jax.ShapeDtypeStruct((T, 1), jnp.float32),
        jax.ShapeDtypeStruct((T, 1), jnp.float32),
        jax.ShapeDtypeStruct((1, 16), jnp.float32),
    ],
)


@functools.partial(
    pl.kernel,
    mesh=_mesh,
    compiler_params=pltpu.CompilerParams(needs_layout_passes=False),
    out_type=[
        jax.ShapeDtypeStruct((E * C,), jnp.int32),
        jax.ShapeDtypeStruct((T,), jnp.int32),
        jax.ShapeDtypeStruct((T,), jnp.int32),
        jax.ShapeDtypeStruct((E * C,), jnp.float32),
        jax.ShapeDtypeStruct((16,), jnp.float32),
    ],
    scratch_types=[
        pltpu.VMEM((T,), jnp.int32),
        pltpu.VMEM((T,), jnp.int32),
        pltpu.VMEM((T,), jnp.float32),
        pltpu.VMEM((T,), jnp.float32),
        pltpu.VMEM((16,), jnp.float32),
        pltpu.VMEM((E * C,), jnp.int32),
        pltpu.VMEM((T,), jnp.int32),
        pltpu.VMEM((T,), jnp.int32),
        pltpu.VMEM((T,), jnp.float32),
        pltpu.VMEM((T,), jnp.float32),
        pltpu.VMEM((E * C,), jnp.float32),
        pltpu.VMEM((16,), jnp.float32),
        pltpu.SemaphoreType.DMA,
    ],
)
def _route(idx1_h, idx2_h, g1_h, g2_h, me_h,
           src_h, slot1_h, slot2_h, wslot_h, laux_h,
           vidx1, vidx2, vg1, vg2, vme, vsrc, vslot1, vslot2, vgw1, vgw2,
           vwslot, vlaux, dsem):
    wid = lax.axis_index("s") * 2 + lax.axis_index("c")

    @pl.when(wid == 0)
    def _():
        hs = [pltpu.async_copy(a, b, dsem) for a, b in
              [(idx1_h, vidx1), (idx2_h, vidx2), (g1_h, vg1),
               (g2_h, vg2), (me_h, vme)]]
        for h in hs:
            h.wait()
        iota16 = lax.iota(jnp.int32, 16)

        def zbody(i, c):
            vsrc[pl.ds(i * 16, 16)] = jnp.zeros((16,), jnp.int32)
            vwslot[pl.ds(i * 16, 16)] = jnp.zeros((16,), jnp.float32)
            return c

        lax.fori_loop(0, (E * C) // 16, zbody, 0)

        def make_pass(vidx, vg, vslot, vgw):
            def body(i, bases):
                ev = vidx[pl.ds(i * 16, 16)]
                gv = vg[pl.ds(i * 16, 16)]
                tvec = i * 16 + iota16
                is_lo = ev < 4
                sh_lo = ev * 8
                sh_hi = (ev - 4) * 8
                one = jnp.int32(1)
                packed_lo = jnp.where(is_lo, one << sh_lo, 0)
                packed_hi = jnp.where(is_lo, 0, one << sh_hi)
                pc_lo = plsc.cumsum(packed_lo)
                pc_hi = plsc.cumsum(packed_hi)
                rank = jnp.where(is_lo, pc_lo >> sh_lo, pc_hi >> sh_hi) & 255
                basev = jnp.zeros((16,), jnp.int32)
                for e in range(E):
                    basev = jnp.where(ev == e, bases[e], basev)
                locv = basev + rank - 1
                tl = pc_lo[15]
                th = pc_hi[15]
                newb = [bases[e] + ((tl >> (8 * e)) & 255) for e in range(4)]
                newb += [bases[4 + e] + ((th >> (8 * e)) & 255)
                         for e in range(4)]
                kept = locv < C
                slotv = jnp.where(kept, ev * C + locv, 0)
                vslot[pl.ds(i * 16, 16)] = slotv
                vgw[pl.ds(i * 16, 16)] = jnp.where(kept, gv, jnp.float32(0.0))
                plsc.store_scatter(vsrc, [slotv], tvec, mask=kept)
                return tuple(newb)
            return body

        zero8 = (jnp.int32(0),) * E
        b1c = lax.fori_loop(0, T // 16, make_pass(vidx1, vg1, vslot1, vgw1),
                            zero8)
        cntv = jnp.zeros((16,), jnp.int32)
        for e in range(E):
            cntv = jnp.where(iota16 == e, b1c[e], cntv)
        s = jnp.sum(vme[...] * cntv.astype(jnp.float32))
        vlaux[...] = jnp.full((16,), jnp.float32(0.0), jnp.float32) + \
            s * jnp.float32(E / T)
        b2c = lax.fori_loop(0, T // 16, make_pass(vidx2, vg2, vslot2, vgw2),
                            b1c)
        dead = jnp.int32(0)
        for e in reversed(range(E)):
            dead = jnp.where(b2c[e] < C, e * C + b2c[e], dead)

        def nbody(i, c):
            a = vgw1[pl.ds(i * 16, 16)]
            b = vgw2[pl.ds(i * 16, 16)]
            den = jnp.maximum(a + b, jnp.float32(1e-9))
            g1n = a / den
            g2n = b / den
            k1 = a > 0
            k2 = b > 0
            s1 = jnp.where(k1, vslot1[pl.ds(i * 16, 16)], dead)
            s2 = jnp.where(k2, vslot2[pl.ds(i * 16, 16)], dead)
            vslot1[pl.ds(i * 16, 16)] = s1
            vslot2[pl.ds(i * 16, 16)] = s2
            plsc.store_scatter(vwslot, [s1], g1n, mask=k1)
            plsc.store_scatter(vwslot, [s2], g2n, mask=k2)
            return c

        lax.fori_loop(0, T // 16, nbody, 0)
        hs2 = [pltpu.async_copy(a, b, dsem) for a, b in
               [(vsrc, src_h), (vslot1, slot1_h), (vslot2, slot2_h),
                (vwslot, wslot_h), (vlaux, laux_h)]]
        for h in hs2:
            h.wait()


_SLOTS_PER_TILE = (E * C) // 32
_DCHUNK = 32
_DN = _SLOTS_PER_TILE // _DCHUNK


@functools.partial(
    pl.kernel,
    mesh=_mesh,
    compiler_params=pltpu.CompilerParams(needs_layout_passes=False),
    out_type=jax.ShapeDtypeStruct((E * C, D), jnp.float32),
    scratch_types=[
        pltpu.VMEM((_SLOTS_PER_TILE,), jnp.int32),
        pltpu.VMEM((_DCHUNK, D), jnp.float32),
        pltpu.VMEM((_DCHUNK, D), jnp.float32),
        pltpu.SemaphoreType.DMA,
        pltpu.SemaphoreType.DMA,
        pltpu.SemaphoreType.DMA,
        pltpu.SemaphoreType.DMA,
    ],
)
def _dispatch(x_h, src_h, xe_h, idxv, rows0, rows1, sg0, sg1, so0, so1):
    wid = lax.axis_index("s") * 2 + lax.axis_index("c")
    base = wid * _SLOTS_PER_TILE
    pltpu.sync_copy(src_h.at[pl.ds(base, _SLOTS_PER_TILE)], idxv)
    rows = [rows0, rows1]
    sg = [sg0, sg1]
    so = [so0, so1]

    def start_gather(ch):
        return pltpu.async_copy(
            x_h.at[idxv.at[pl.ds(ch * _DCHUNK, _DCHUNK)]],
            rows[ch % 2], sg[ch % 2])

    gh = [start_gather(0), start_gather(1)]
    sh = [None] * _DN
    for ch in range(_DN):
        gh[ch % 2].wait()
        sh[ch] = pltpu.async_copy(
            rows[ch % 2], xe_h.at[pl.ds(base + ch * _DCHUNK, _DCHUNK)],
            so[ch % 2])
        if ch + 2 < _DN:
            sh[ch].wait()
            gh[ch % 2] = start_gather(ch + 2)
    for ch in (_DN - 2, _DN - 1):
        sh[ch].wait()


def _ffn_body(xe_ref, w1_ref, b1_ref, w2_ref, b2_ref, ws_ref, eo_ref):
    xv = xe_ref[...]
    h = jnp.dot(xv, w1_ref[0], preferred_element_type=jnp.float32) + \
        b1_ref[0]
    h = jnp.maximum(h, 0.0)
    o = jnp.dot(h, w2_ref[0], preferred_element_type=jnp.float32) + \
        b2_ref[0]
    eo_ref[...] = o * ws_ref[...]


_ffn = pl.pallas_call(
    _ffn_body,
    grid=(E,),
    in_specs=[
        pl.BlockSpec((C, D), lambda e: (e, 0)),
        pl.BlockSpec((1, D, F), lambda e: (e, 0, 0)),
        pl.BlockSpec((1, 1, F), lambda e: (e, 0, 0)),
        pl.BlockSpec((1, F, D), lambda e: (e, 0, 0)),
        pl.BlockSpec((1, 1, D), lambda e: (e, 0, 0)),
        pl.BlockSpec((C, 1), lambda e: (e, 0)),
    ],
    out_specs=pl.BlockSpec((C, D), lambda e: (e, 0)),
    out_shape=jax.ShapeDtypeStruct((E * C, D), jnp.float32),
    compiler_params=pltpu.CompilerParams(
        dimension_semantics=("arbitrary",)),
)


_TOK_PER_TILE = T // 32
_CCHUNK = 16
_CN = _TOK_PER_TILE // _CCHUNK


@functools.partial(
    pl.kernel,
    mesh=_mesh,
    compiler_params=pltpu.CompilerParams(needs_layout_passes=False),
    out_type=jax.ShapeDtypeStruct((T, D), jnp.float32),
    scratch_types=[
        pltpu.VMEM((_TOK_PER_TILE,), jnp.int32),
        pltpu.VMEM((_TOK_PER_TILE,), jnp.int32),
        pltpu.VMEM((_CCHUNK, D), jnp.float32),
        pltpu.VMEM((_CCHUNK, D), jnp.float32),
        pltpu.VMEM((_CCHUNK, D), jnp.float32),
        pltpu.VMEM((_CCHUNK, D), jnp.float32),
        pltpu.SemaphoreType.DMA,
        pltpu.SemaphoreType.DMA,
        pltpu.SemaphoreType.DMA,
        pltpu.SemaphoreType.DMA,
    ],
)
def _combine(eo_h, slot1_h, slot2_h, out_h,
             s1v, s2v, r1a, r1b, r2a, r2b, sg0, sg1, so0, so1):
    wid = lax.axis_index("s") * 2 + lax.axis_index("c")
    base = wid * _TOK_PER_TILE
    pltpu.sync_copy(slot1_h.at[pl.ds(base, _TOK_PER_TILE)], s1v)
    pltpu.sync_copy(slot2_h.at[pl.ds(base, _TOK_PER_TILE)], s2v)
    r1 = [r1a, r1b]
    r2 = [r2a, r2b]
    sg = [sg0, sg1]
    so = [so0, so1]

    def start_gather(ch):
        p = ch % 2
        h1 = pltpu.async_copy(
            eo_h.at[s1v.at[pl.ds(ch * _CCHUNK, _CCHUNK)]], r1[p], sg[p])
        h2 = pltpu.async_copy(
            eo_h.at[s2v.at[pl.ds(ch * _CCHUNK, _CCHUNK)]], r2[p], sg[p])
        return (h1, h2)

    gh = [start_gather(0), start_gather(1)]
    sh = [None] * _CN
    for ch in range(_CN):
        p = ch % 2
        gh[p][0].wait()
        gh[p][1].wait()

        for j in range(_CCHUNK):
            def cbj(k, c, p=p, j=j):
                for u in range(4):
                    sl = pl.ds(k * 64 + u * 16, 16)
                    plsc.addupdate(r1[p].at[j, sl], r2[p][j, sl])
                return c
            lax.fori_loop(0, D // 64, cbj, 0)
        sh[ch] = pltpu.async_copy(
            r1[p], out_h.at[pl.ds(base + ch * _CCHUNK, _CCHUNK)], so[p])
        if ch + 2 < _CN:
            sh[ch].wait()
            gh[p] = start_gather(ch + 2)
    sh[_CN - 2].wait()
    sh[_CN - 1].wait()


def kernel(x, wg, w1, b1, w2, b2):
    xt = x.reshape(T, D)
    wgp = jnp.pad(wg, ((0, 0), (0, 128 - E)))
    i1, i2, g1r, g2r, me = _gate(xt, wgp)
    src, slot1, slot2, wslot, laux = _route(
        i1.reshape(T), i2.reshape(T), g1r.reshape(T), g2r.reshape(T),
        me.reshape(16))
    xe = _dispatch(xt, src)
    eo = _ffn(xe, w1, b1.reshape(E, 1, F), w2, b2.reshape(E, 1, D),
              wslot.reshape(E * C, 1))
    out = _combine(eo, slot1, slot2)
    return out.reshape(x.shape), laux[0]
